# Initial kernel scaffold; baseline (speedup 1.0000x reference)
import jax
import jax.numpy as jnp
from jax.experimental import pallas as pl


def kernel(x, edge_index, edge_attr, params):
    raise NotImplementedError("write your pallas kernel here")


# speedup vs baseline: 7.5356x; 7.5356x over previous
import functools

import jax
import jax.numpy as jnp
from jax import lax
from jax.experimental import pallas as pl
from jax.experimental.pallas import tpu as pltpu
from jax.experimental.pallas import tpu_sc as plsc

N = 10000
E = 320000
D = 128
NL = 3
NC, NS = 2, 16
NW = NC * NS
EPT = E // NW
IW = 80
KI = 5
CH = KI * IW
NCHUNK = EPT // CH
STRIPE = 664
TAIL = N - (NS - 1) * STRIPE


def _sc_msg_body(h2_hbm, src_hbm, dst_hbm, ea0_hbm, ea1_hbm, ea2_hbm,
                 zeros_hbm, out_hbm, aggr_s, srcv, ea0v, ea1v, ea2v, gidx,
                 rows, sem, *dstv):
    c = lax.axis_index("c")
    s = lax.axis_index("s")
    wid = s * NC + c

    @pl.when(s < NS - 1)
    def _():
        pltpu.sync_copy(zeros_hbm, aggr_s.at[pl.ds(s * STRIPE, STRIPE)])

    @pl.when(s == NS - 1)
    def _():
        pltpu.sync_copy(zeros_hbm.at[pl.ds(0, TAIL)],
                        aggr_s.at[pl.ds((NS - 1) * STRIPE, TAIL)])

    plsc.subcore_barrier()

    def chunk_body(m, carry):
        base = wid * EPT + m * CH
        pltpu.sync_copy(src_hbm.at[pl.ds(base, CH)], srcv)
        pltpu.sync_copy(ea0_hbm.at[pl.ds(base, CH)], ea0v)
        pltpu.sync_copy(ea1_hbm.at[pl.ds(base, CH)], ea1v)
        pltpu.sync_copy(ea2_hbm.at[pl.ds(base, CH)], ea2v)
        for j in range(KI):
            pltpu.sync_copy(dst_hbm.at[pl.ds(base + j * IW, IW)], dstv[j])
        for g in range(CH // 16):
            sl = pl.ds(g * 16, 16)
            gidx[sl] = (srcv[sl] * 8 + ea0v[sl] * 4
                        + ea1v[sl] * 2 + ea2v[sl])

        for j in range(KI):
            pltpu.async_copy(h2_hbm.at[gidx.at[pl.ds(j * IW, IW)]],
                             rows, sem).wait()

            def row_body(r, carry2):
                for q in range(D // 16):
                    sl = pl.ds(q * 16, 16)
                    rows[r, sl] = jnp.maximum(rows[r, sl], 0.0)
                return carry2
            lax.fori_loop(0, IW, row_body, 0)

            pltpu.sync_copy(rows, aggr_s.at[dstv[j]], add=True)
        return carry

    lax.fori_loop(0, NCHUNK, chunk_body, 0)
    plsc.subcore_barrier()

    @pl.when(s < NS - 1)
    def _():
        sl = pl.ds(s * STRIPE, STRIPE)
        pltpu.sync_copy(aggr_s.at[sl], out_hbm.at[c, sl])

    @pl.when(s == NS - 1)
    def _():
        sl = pl.ds((NS - 1) * STRIPE, TAIL)
        pltpu.sync_copy(aggr_s.at[sl], out_hbm.at[c, sl])


@functools.cache
def _get_sc_msg():
  return pl.kernel(
    _sc_msg_body,
    out_type=jax.ShapeDtypeStruct((NC, N, D), jnp.float32),
    mesh=plsc.VectorSubcoreMesh(core_axis_name="c", subcore_axis_name="s"),
    scratch_types=[
        pltpu.VMEM_SHARED((N, D), jnp.float32),
        pltpu.VMEM((CH,), jnp.int32),
        pltpu.VMEM((CH,), jnp.int32),
        pltpu.VMEM((CH,), jnp.int32),
        pltpu.VMEM((CH,), jnp.int32),
        pltpu.VMEM((CH,), jnp.int32),
        pltpu.VMEM((IW, D), jnp.float32),
        pltpu.SemaphoreType.DMA,
    ] + [pltpu.VMEM((IW,), jnp.int32) for _ in range(KI)],
    name="gin_msg_sc",
  )


def _expand_body(h_ref, combos_ref, out_ref):
    out_ref[...] = h_ref[...][:, None, :] + combos_ref[...][None, :, :]


_EXB = 1000
_expand = pl.pallas_call(
    _expand_body,
    grid=(N // _EXB,),
    in_specs=[
        pl.BlockSpec((_EXB, D), lambda i: (i, 0)),
        pl.BlockSpec((8, D), lambda i: (0, 0)),
    ],
    out_specs=pl.BlockSpec((_EXB, 8, D), lambda i: (i, 0, 0)),
    out_shape=jax.ShapeDtypeStruct((N, 8, D), jnp.float32),
    name="gin_expand_tc",
)


def _enc_body(xf_ref, a0_ref, a1_ref, bond0_ref, bond1_ref,
              h0_ref, combos_ref):
    xf = xf_ref[...]
    h = jnp.zeros((N, D), jnp.float32)
    for i in range(9):
        cond = xf[:, i:i + 1] > 0.5
        h = h + jnp.where(cond, a1_ref[i], a0_ref[i])
    h0_ref[...] = h
    for l in range(NL):
        rows = []
        for code in range(8):
            t0 = bond1_ref[l, 0] if code & 4 else bond0_ref[l, 0]
            t1 = bond1_ref[l, 1] if code & 2 else bond0_ref[l, 1]
            t2 = bond1_ref[l, 2] if code & 1 else bond0_ref[l, 2]
            rows.append((t0 + t1) + t2)
        combos_ref[l] = jnp.stack(rows)


_enc = pl.pallas_call(
    _enc_body,
    out_shape=(
        jax.ShapeDtypeStruct((N, D), jnp.float32),
        jax.ShapeDtypeStruct((NL, 8, D), jnp.float32),
    ),
    name="gin_encoder_tc",
)


def _mlp_body(relu_out, h_ref, a0_ref, a1_ref, eps_ref, w1_ref, b1_ref,
              g1_ref, t1_ref, w2_ref, b2_ref, g2_ref, t2_ref, out_ref):
    z = (1.0 + eps_ref[0, 0]) * h_ref[...] + a0_ref[...] + a1_ref[...]
    u = jnp.dot(z, w1_ref[...], preferred_element_type=jnp.float32)
    u = u + b1_ref[...]
    m = jnp.mean(u, axis=0, keepdims=True)
    u = u - m
    v = jnp.mean(u * u, axis=0, keepdims=True)
    u = u * lax.rsqrt(v + 1e-5) * g1_ref[...] + t1_ref[...]
    u = jnp.maximum(u, 0.0)
    w = jnp.dot(u, w2_ref[...], preferred_element_type=jnp.float32)
    w = w + b2_ref[...]
    m2 = jnp.mean(w, axis=0, keepdims=True)
    w = w - m2
    v2 = jnp.mean(w * w, axis=0, keepdims=True)
    w = w * lax.rsqrt(v2 + 1e-5) * g2_ref[...] + t2_ref[...]
    if relu_out:
        w = jnp.maximum(w, 0.0)
    out_ref[...] = w


_mlp_relu = pl.pallas_call(
    functools.partial(_mlp_body, True),
    out_shape=jax.ShapeDtypeStruct((N, D), jnp.float32),
    name="gin_mlp_relu_tc",
)
_mlp_last = pl.pallas_call(
    functools.partial(_mlp_body, False),
    out_shape=jax.ShapeDtypeStruct((N, D), jnp.float32),
    name="gin_mlp_last_tc",
)


def kernel(x, edge_index, edge_attr, params):
    xf = jnp.pad(x.astype(jnp.float32), ((0, 0), (0, 16 - x.shape[1])))
    a0 = jnp.pad(jnp.stack([t[0] for t in params["atom_tables"]]),
                 ((0, 16 - len(params["atom_tables"])), (0, 0)))
    a1 = jnp.pad(jnp.stack([t[1] for t in params["atom_tables"]]),
                 ((0, 16 - len(params["atom_tables"])), (0, 0)))
    layers = params["layers"]
    bond0 = jnp.stack([jnp.stack([t[0] for t in p["bond_tables"]])
                       for p in layers])
    bond1 = jnp.stack([jnp.stack([t[1] for t in p["bond_tables"]])
                       for p in layers])

    h0, combos = _enc(xf, a0, a1, bond0, bond1)

    src = edge_index[0]
    dst = edge_index[1]
    ea0, ea1, ea2 = edge_attr[:, 0], edge_attr[:, 1], edge_attr[:, 2]
    zeros = jnp.zeros((STRIPE, D), jnp.float32)

    h = h0
    for l, p in enumerate(layers):
        h2 = _expand(h, combos[l]).reshape(8 * N, D)
        aggr2 = _get_sc_msg()(h2, src, dst, ea0, ea1, ea2, zeros)
        mlp = _mlp_relu if l < NL - 1 else _mlp_last
        h = mlp(h, aggr2[0], aggr2[1],
                p["eps"].reshape(1, 1),
                p["W1"], p["b1"].reshape(1, -1), p["g1"].reshape(1, -1),
                p["bt1"].reshape(1, -1),
                p["W2"], p["b2"].reshape(1, -1), p["bn_g"].reshape(1, -1),
                p["bn_b"].reshape(1, -1))
    return h


# speedup vs baseline: 17.9630x; 2.3837x over previous
import functools

import jax
import jax.numpy as jnp
from jax import lax
from jax.experimental import pallas as pl
from jax.experimental.pallas import tpu as pltpu
from jax.experimental.pallas import tpu_sc as plsc

N = 10000
E = 320000
D = 128
NL = 3
NC, NS = 2, 16
NW = NC * NS
EPT = E // NW
STRIPE = 664
TAIL = N - (NS - 1) * STRIPE
SW = 40
NSUB = EPT // SW
RING = 5
NCYC = NSUB // RING


def _sc_msg_body(h2_hbm, gidx_hbm, dst_hbm, zeros_hbm, out_hbm,
                 aggr_s, gidx_t, semi, semg, sems, b0, b1, b2, b3, b4,
                 d0, d1, d2, d3, d4):
    bufs = (b0, b1, b2, b3, b4)
    dstb = (d0, d1, d2, d3, d4)
    c = lax.axis_index("c")
    s = lax.axis_index("s")
    wid = s * NC + c
    tbase = wid * EPT

    pltpu.sync_copy(gidx_hbm.at[pl.ds(tbase, EPT)], gidx_t)

    @pl.when(s < NS - 1)
    def _():
        pltpu.sync_copy(zeros_hbm, aggr_s.at[pl.ds(s * STRIPE, STRIPE)])

    @pl.when(s == NS - 1)
    def _():
        pltpu.sync_copy(zeros_hbm.at[pl.ds(0, TAIL)],
                        aggr_s.at[pl.ds((NS - 1) * STRIPE, TAIL)])

    plsc.subcore_barrier()

    def fire(k, b):
        pltpu.async_copy(dst_hbm.at[pl.ds(tbase + k * SW, SW)], dstb[b],
                         semi.at[b])
        pltpu.async_copy(h2_hbm.at[gidx_t.at[pl.ds(k * SW, SW)]], bufs[b],
                         semg.at[b])

    def wait_in(k, b):
        pltpu.make_async_copy(dst_hbm.at[pl.ds(tbase + k * SW, SW)],
                              dstb[b], semi.at[b]).wait()
        pltpu.make_async_copy(h2_hbm.at[gidx_t.at[pl.ds(k * SW, SW)]],
                              bufs[b], semg.at[b]).wait()

    def fire_s(b):
        pltpu.async_copy(bufs[b], aggr_s.at[dstb[b]], sems.at[b], add=True)

    def wait_s(b):
        pltpu.make_async_copy(bufs[b], aggr_s.at[dstb[b]], sems.at[b]).wait()

    fire(0, 0)
    fire(1, 1)

    def cyc_body(t, carry):
        k0 = t * RING
        for i in range(RING):
            k = k0 + i
            bf = (i + 2) % RING

            @pl.when(k >= 3)
            def _(bf=bf):
                wait_s(bf)

            @pl.when(k + 2 < NSUB)
            def _(k=k, bf=bf):
                fire(k + 2, bf)

            wait_in(k, i)
            fire_s(i)
        return carry

    lax.fori_loop(0, NCYC, cyc_body, 0)
    for j in range(NSUB - 3, NSUB):
        wait_s(j % RING)
    plsc.subcore_barrier()

    @pl.when(s < NS - 1)
    def _():
        sl = pl.ds(s * STRIPE, STRIPE)
        pltpu.sync_copy(aggr_s.at[sl], out_hbm.at[c, sl])

    @pl.when(s == NS - 1)
    def _():
        sl = pl.ds((NS - 1) * STRIPE, TAIL)
        pltpu.sync_copy(aggr_s.at[sl], out_hbm.at[c, sl])


@functools.cache
def _get_sc_msg():
  return pl.kernel(
    _sc_msg_body,
    out_type=jax.ShapeDtypeStruct((NC, N, D), jnp.float32),
    mesh=plsc.VectorSubcoreMesh(core_axis_name="c", subcore_axis_name="s"),
    scratch_types=[
        pltpu.VMEM_SHARED((N, D), jnp.float32),
        pltpu.VMEM((EPT,), jnp.int32),
        pltpu.SemaphoreType.DMA((RING,)),
        pltpu.SemaphoreType.DMA((RING,)),
        pltpu.SemaphoreType.DMA((RING,)),
    ] + [pltpu.VMEM((SW, D), jnp.float32) for _ in range(RING)]
      + [pltpu.VMEM((SW,), jnp.int32) for _ in range(RING)],
    name="gin_msg_sc",
  )


def _expand_body(h_ref, combos_ref, out_ref):
    out_ref[...] = jnp.maximum(
        h_ref[...][:, None, :] + combos_ref[...][None, :, :], 0.0)


_EXB = 1000
_expand = pl.pallas_call(
    _expand_body,
    grid=(N // _EXB,),
    in_specs=[
        pl.BlockSpec((_EXB, D), lambda i: (i, 0)),
        pl.BlockSpec((8, D), lambda i: (0, 0)),
    ],
    out_specs=pl.BlockSpec((_EXB, 8, D), lambda i: (i, 0, 0)),
    out_shape=jax.ShapeDtypeStruct((N, 8, D), jnp.float32),
    name="gin_expand_tc",
)


def _enc_body(xf_ref, a0_ref, a1_ref, bond0_ref, bond1_ref, src_ref,
              e0_ref, e1_ref, e2_ref, h0_ref, combos_ref, gidx_ref):
    gidx_ref[...] = (src_ref[...] * 8 + e0_ref[...] * 4
                     + e1_ref[...] * 2 + e2_ref[...])
    xf = xf_ref[...]
    h = jnp.zeros((N, D), jnp.float32)
    for i in range(9):
        cond = xf[:, i:i + 1] > 0.5
        h = h + jnp.where(cond, a1_ref[i], a0_ref[i])
    h0_ref[...] = h
    for l in range(NL):
        rows = []
        for code in range(8):
            t0 = bond1_ref[l, 0] if code & 4 else bond0_ref[l, 0]
            t1 = bond1_ref[l, 1] if code & 2 else bond0_ref[l, 1]
            t2 = bond1_ref[l, 2] if code & 1 else bond0_ref[l, 2]
            rows.append((t0 + t1) + t2)
        combos_ref[l] = jnp.stack(rows)


_enc = pl.pallas_call(
    _enc_body,
    out_shape=(
        jax.ShapeDtypeStruct((N, D), jnp.float32),
        jax.ShapeDtypeStruct((NL, 8, D), jnp.float32),
        jax.ShapeDtypeStruct((E // 128, 128), jnp.int32),
    ),
    name="gin_encoder_tc",
)


def _mlp_body(relu_out, h_ref, a0_ref, a1_ref, eps_ref, w1_ref, b1_ref,
              g1_ref, t1_ref, w2_ref, b2_ref, g2_ref, t2_ref, out_ref):
    z = (1.0 + eps_ref[0, 0]) * h_ref[...] + a0_ref[...] + a1_ref[...]
    u = jnp.dot(z, w1_ref[...], preferred_element_type=jnp.float32)
    u = u + b1_ref[...]
    m = jnp.mean(u, axis=0, keepdims=True)
    u = u - m
    v = jnp.mean(u * u, axis=0, keepdims=True)
    u = u * lax.rsqrt(v + 1e-5) * g1_ref[...] + t1_ref[...]
    u = jnp.maximum(u, 0.0)
    w = jnp.dot(u, w2_ref[...], preferred_element_type=jnp.float32)
    w = w + b2_ref[...]
    m2 = jnp.mean(w, axis=0, keepdims=True)
    w = w - m2
    v2 = jnp.mean(w * w, axis=0, keepdims=True)
    w = w * lax.rsqrt(v2 + 1e-5) * g2_ref[...] + t2_ref[...]
    if relu_out:
        w = jnp.maximum(w, 0.0)
    out_ref[...] = w


_mlp_relu = pl.pallas_call(
    functools.partial(_mlp_body, True),
    out_shape=jax.ShapeDtypeStruct((N, D), jnp.float32),
    name="gin_mlp_relu_tc",
)
_mlp_last = pl.pallas_call(
    functools.partial(_mlp_body, False),
    out_shape=jax.ShapeDtypeStruct((N, D), jnp.float32),
    name="gin_mlp_last_tc",
)


def kernel(x, edge_index, edge_attr, params):
    xf = jnp.pad(x.astype(jnp.float32), ((0, 0), (0, 16 - x.shape[1])))
    a0 = jnp.pad(jnp.stack([t[0] for t in params["atom_tables"]]),
                 ((0, 16 - len(params["atom_tables"])), (0, 0)))
    a1 = jnp.pad(jnp.stack([t[1] for t in params["atom_tables"]]),
                 ((0, 16 - len(params["atom_tables"])), (0, 0)))
    layers = params["layers"]
    bond0 = jnp.stack([jnp.stack([t[0] for t in p["bond_tables"]])
                       for p in layers])
    bond1 = jnp.stack([jnp.stack([t[1] for t in p["bond_tables"]])
                       for p in layers])

    src = edge_index[0]
    dst = edge_index[1]
    h0, combos, gidx2d = _enc(
        xf, a0, a1, bond0, bond1, src.reshape(E // 128, 128),
        edge_attr[:, 0].reshape(E // 128, 128),
        edge_attr[:, 1].reshape(E // 128, 128),
        edge_attr[:, 2].reshape(E // 128, 128))
    gidx = gidx2d.reshape(E)
    zeros = jnp.zeros((STRIPE, D), jnp.float32)

    h = h0
    for l, p in enumerate(layers):
        h2 = _expand(h, combos[l]).reshape(8 * N, D)
        aggr2 = _get_sc_msg()(h2, gidx, dst, zeros)
        mlp = _mlp_relu if l < NL - 1 else _mlp_last
        h = mlp(h, aggr2[0], aggr2[1],
                p["eps"].reshape(1, 1),
                p["W1"], p["b1"].reshape(1, -1), p["g1"].reshape(1, -1),
                p["bt1"].reshape(1, -1),
                p["W2"], p["b2"].reshape(1, -1), p["bn_g"].reshape(1, -1),
                p["bn_b"].reshape(1, -1))
    return h


# speedup vs baseline: 18.7211x; 1.0422x over previous
import functools

import jax
import jax.numpy as jnp
from jax import lax
from jax.experimental import pallas as pl
from jax.experimental.pallas import tpu as pltpu
from jax.experimental.pallas import tpu_sc as plsc

N = 10000
E = 320000
D = 128
NL = 3
NC, NS = 2, 16
NW = NC * NS
EPT = E // NW
STRIPE = 664
TAIL = N - (NS - 1) * STRIPE
SW = 80
NSUB = EPT // SW
RING = 3
NCYC = (NSUB - 2) // RING


def _sc_msg_body(h2_hbm, packed_hbm, zeros_hbm, out_hbm,
                 aggr_s, packed_t, semg, sems, b0, b1, b2,
                 g0, g1, g2, d0, d1, d2):
    bufs = (b0, b1, b2)
    gidxb = (g0, g1, g2)
    dstb = (d0, d1, d2)
    c = lax.axis_index("c")
    s = lax.axis_index("s")
    wid = s * NC + c
    tbase = wid * EPT

    pltpu.sync_copy(packed_hbm.at[pl.ds(tbase, EPT)], packed_t)

    @pl.when(s < NS - 1)
    def _():
        pltpu.sync_copy(zeros_hbm, aggr_s.at[pl.ds(s * STRIPE, STRIPE)])

    @pl.when(s == NS - 1)
    def _():
        pltpu.sync_copy(zeros_hbm.at[pl.ds(0, TAIL)],
                        aggr_s.at[pl.ds((NS - 1) * STRIPE, TAIL)])

    plsc.subcore_barrier()

    def unpack(k, b):
        for g in range(SW // 16):
            sl = pl.ds(g * 16, 16)
            w = packed_t[pl.ds(k * SW + g * 16, 16)]
            gidxb[b][sl] = lax.bitwise_and(w, 0x1FFFF)
            dstb[b][sl] = lax.shift_right_logical(w, 17)

    def fire_g(k, b):
        pltpu.async_copy(h2_hbm.at[gidxb[b]], bufs[b], semg.at[b])

    def wait_g(b):
        pltpu.make_async_copy(h2_hbm.at[gidxb[b]], bufs[b],
                              semg.at[b]).wait()

    def fire_s(b):
        pltpu.async_copy(bufs[b], aggr_s.at[dstb[b]], sems.at[b], add=True)

    def wait_s(b):
        pltpu.make_async_copy(bufs[b], aggr_s.at[dstb[b]], sems.at[b]).wait()

    def step(k, i):
        bf = (i + 1) % RING

        @pl.when(k >= 2)
        def _():
            wait_s(bf)

        @pl.when(k + 1 < NSUB)
        def _():
            unpack(k + 1, bf)
            fire_g(k + 1, bf)

        wait_g(i)
        fire_s(i)

    unpack(0, 0)
    fire_g(0, 0)

    def cyc_body(t, carry):
        k0 = t * RING
        for i in range(RING):
            step(k0 + i, i)
        return carry

    lax.fori_loop(0, NCYC, cyc_body, 0)
    for k in range(NCYC * RING, NSUB):
        step(jnp.int32(k), k % RING)
    for k in range(NSUB - 2, NSUB):
        wait_s(k % RING)
    plsc.subcore_barrier()

    @pl.when(s < NS - 1)
    def _():
        sl = pl.ds(s * STRIPE, STRIPE)
        pltpu.sync_copy(aggr_s.at[sl], out_hbm.at[c, sl])

    @pl.when(s == NS - 1)
    def _():
        sl = pl.ds((NS - 1) * STRIPE, TAIL)
        pltpu.sync_copy(aggr_s.at[sl], out_hbm.at[c, sl])


@functools.cache
def _get_sc_msg():
  return pl.kernel(
    _sc_msg_body,
    out_type=jax.ShapeDtypeStruct((NC, N, D), jnp.float32),
    mesh=plsc.VectorSubcoreMesh(core_axis_name="c", subcore_axis_name="s"),
    scratch_types=[
        pltpu.VMEM_SHARED((N, D), jnp.float32),
        pltpu.VMEM((EPT,), jnp.int32),
        pltpu.SemaphoreType.DMA((RING,)),
        pltpu.SemaphoreType.DMA((RING,)),
    ] + [pltpu.VMEM((SW, D), jnp.float32) for _ in range(RING)]
      + [pltpu.VMEM((SW,), jnp.int32) for _ in range(RING)]
      + [pltpu.VMEM((SW,), jnp.int32) for _ in range(RING)],
    name="gin_msg_sc",
  )


def _expand_body(h_ref, combos_ref, out_ref):
    out_ref[...] = jnp.maximum(
        h_ref[...][:, None, :] + combos_ref[...][None, :, :], 0.0)


_EXB = 1000
_expand = pl.pallas_call(
    _expand_body,
    grid=(N // _EXB,),
    in_specs=[
        pl.BlockSpec((_EXB, D), lambda i: (i, 0)),
        pl.BlockSpec((8, D), lambda i: (0, 0)),
    ],
    out_specs=pl.BlockSpec((_EXB, 8, D), lambda i: (i, 0, 0)),
    out_shape=jax.ShapeDtypeStruct((N, 8, D), jnp.float32),
    name="gin_expand_tc",
)


def _enc_body(xf_ref, a0_ref, a1_ref, bond0_ref, bond1_ref, src_ref,
              dst_ref, e0_ref, e1_ref, e2_ref, h0_ref, combos_ref,
              packed_ref):
    gidx = (src_ref[...] * 8 + e0_ref[...] * 4
            + e1_ref[...] * 2 + e2_ref[...])
    packed_ref[...] = lax.bitwise_or(gidx, lax.shift_left(dst_ref[...], 17))
    xf = xf_ref[...]
    h = jnp.zeros((N, D), jnp.float32)
    for i in range(9):
        cond = xf[:, i:i + 1] > 0.5
        h = h + jnp.where(cond, a1_ref[i], a0_ref[i])
    h0_ref[...] = h
    for l in range(NL):
        rows = []
        for code in range(8):
            t0 = bond1_ref[l, 0] if code & 4 else bond0_ref[l, 0]
            t1 = bond1_ref[l, 1] if code & 2 else bond0_ref[l, 1]
            t2 = bond1_ref[l, 2] if code & 1 else bond0_ref[l, 2]
            rows.append((t0 + t1) + t2)
        combos_ref[l] = jnp.stack(rows)


_enc = pl.pallas_call(
    _enc_body,
    out_shape=(
        jax.ShapeDtypeStruct((N, D), jnp.float32),
        jax.ShapeDtypeStruct((NL, 8, D), jnp.float32),
        jax.ShapeDtypeStruct((E // 128, 128), jnp.int32),
    ),
    name="gin_encoder_tc",
)


def _mlp_body(relu_out, h_ref, a0_ref, a1_ref, eps_ref, w1_ref, b1_ref,
              g1_ref, t1_ref, w2_ref, b2_ref, g2_ref, t2_ref, out_ref):
    z = (1.0 + eps_ref[0, 0]) * h_ref[...] + a0_ref[...] + a1_ref[...]
    u = jnp.dot(z, w1_ref[...], preferred_element_type=jnp.float32)
    u = u + b1_ref[...]
    m = jnp.mean(u, axis=0, keepdims=True)
    u = u - m
    v = jnp.mean(u * u, axis=0, keepdims=True)
    u = u * lax.rsqrt(v + 1e-5) * g1_ref[...] + t1_ref[...]
    u = jnp.maximum(u, 0.0)
    w = jnp.dot(u, w2_ref[...], preferred_element_type=jnp.float32)
    w = w + b2_ref[...]
    m2 = jnp.mean(w, axis=0, keepdims=True)
    w = w - m2
    v2 = jnp.mean(w * w, axis=0, keepdims=True)
    w = w * lax.rsqrt(v2 + 1e-5) * g2_ref[...] + t2_ref[...]
    if relu_out:
        w = jnp.maximum(w, 0.0)
    out_ref[...] = w


_mlp_relu = pl.pallas_call(
    functools.partial(_mlp_body, True),
    out_shape=jax.ShapeDtypeStruct((N, D), jnp.float32),
    name="gin_mlp_relu_tc",
)
_mlp_last = pl.pallas_call(
    functools.partial(_mlp_body, False),
    out_shape=jax.ShapeDtypeStruct((N, D), jnp.float32),
    name="gin_mlp_last_tc",
)


def kernel(x, edge_index, edge_attr, params):
    xf = jnp.pad(x.astype(jnp.float32), ((0, 0), (0, 16 - x.shape[1])))
    a0 = jnp.pad(jnp.stack([t[0] for t in params["atom_tables"]]),
                 ((0, 16 - len(params["atom_tables"])), (0, 0)))
    a1 = jnp.pad(jnp.stack([t[1] for t in params["atom_tables"]]),
                 ((0, 16 - len(params["atom_tables"])), (0, 0)))
    layers = params["layers"]
    bond0 = jnp.stack([jnp.stack([t[0] for t in p["bond_tables"]])
                       for p in layers])
    bond1 = jnp.stack([jnp.stack([t[1] for t in p["bond_tables"]])
                       for p in layers])

    h0, combos, packed2d = _enc(
        xf, a0, a1, bond0, bond1,
        edge_index[0].reshape(E // 128, 128),
        edge_index[1].reshape(E // 128, 128),
        edge_attr[:, 0].reshape(E // 128, 128),
        edge_attr[:, 1].reshape(E // 128, 128),
        edge_attr[:, 2].reshape(E // 128, 128))
    packed = packed2d.reshape(E)
    zeros = jnp.zeros((STRIPE, D), jnp.float32)

    h = h0
    for l, p in enumerate(layers):
        h2 = _expand(h, combos[l]).reshape(8 * N, D)
        aggr2 = _get_sc_msg()(h2, packed, zeros)
        mlp = _mlp_relu if l < NL - 1 else _mlp_last
        h = mlp(h, aggr2[0], aggr2[1],
                p["eps"].reshape(1, 1),
                p["W1"], p["b1"].reshape(1, -1), p["g1"].reshape(1, -1),
                p["bt1"].reshape(1, -1),
                p["W2"], p["b2"].reshape(1, -1), p["bn_g"].reshape(1, -1),
                p["bn_b"].reshape(1, -1))
    return h


# speedup vs baseline: 18.8270x; 1.0057x over previous
import functools

import jax
import jax.numpy as jnp
from jax import lax
from jax.experimental import pallas as pl
from jax.experimental.pallas import tpu as pltpu
from jax.experimental.pallas import tpu_sc as plsc

N = 10000
E = 320000
D = 128
NL = 3
NC, NS = 2, 16
NW = NC * NS
EPT = E // NW
STRIPE = 664
TAIL = N - (NS - 1) * STRIPE
SW = 80
NSUB = EPT // SW
RING = 3
NCYC = (NSUB - 2) // RING


def _sc_msg_body(h2_hbm, packed_hbm, zeros_hbm, out_hbm,
                 aggr_s, packed_t, semg, sems, b0, b1, b2,
                 g0, g1, g2, d0, d1, d2):
    bufs = (b0, b1, b2)
    gidxb = (g0, g1, g2)
    dstb = (d0, d1, d2)
    c = lax.axis_index("c")
    s = lax.axis_index("s")
    wid = s * NC + c
    tbase = wid * EPT

    stg = pltpu.async_copy(packed_hbm.at[pl.ds(tbase, EPT)], packed_t,
                           semg.at[0])

    @pl.when(s < NS - 1)
    def _():
        pltpu.sync_copy(zeros_hbm, aggr_s.at[pl.ds(s * STRIPE, STRIPE)])

    @pl.when(s == NS - 1)
    def _():
        pltpu.sync_copy(zeros_hbm.at[pl.ds(0, TAIL)],
                        aggr_s.at[pl.ds((NS - 1) * STRIPE, TAIL)])

    stg.wait()
    plsc.subcore_barrier()

    def unpack(k, b):
        for g in range(SW // 16):
            sl = pl.ds(g * 16, 16)
            w = packed_t[pl.ds(k * SW + g * 16, 16)]
            gidxb[b][sl] = lax.bitwise_and(w, 0x1FFFF)
            dstb[b][sl] = lax.shift_right_logical(w, 17)

    def fire_g(k, b):
        pltpu.async_copy(h2_hbm.at[gidxb[b]], bufs[b], semg.at[b])

    def wait_g(b):
        pltpu.make_async_copy(h2_hbm.at[gidxb[b]], bufs[b],
                              semg.at[b]).wait()

    def fire_s(b):
        pltpu.async_copy(bufs[b], aggr_s.at[dstb[b]], sems.at[b], add=True)

    def wait_s(b):
        pltpu.make_async_copy(bufs[b], aggr_s.at[dstb[b]], sems.at[b]).wait()

    def step(k, i):
        bf = (i + 1) % RING

        @pl.when(k >= 2)
        def _():
            wait_s(bf)

        @pl.when(k + 1 < NSUB)
        def _():
            unpack(k + 1, bf)
            fire_g(k + 1, bf)

        wait_g(i)
        fire_s(i)

    unpack(0, 0)
    fire_g(0, 0)

    def cyc_body(t, carry):
        k0 = t * RING
        for i in range(RING):
            step(k0 + i, i)
        return carry

    lax.fori_loop(0, NCYC, cyc_body, 0)
    for k in range(NCYC * RING, NSUB):
        step(jnp.int32(k), k % RING)
    for k in range(NSUB - 2, NSUB):
        wait_s(k % RING)
    plsc.subcore_barrier()

    @pl.when(s < NS - 1)
    def _():
        sl = pl.ds(s * STRIPE, STRIPE)
        pltpu.sync_copy(aggr_s.at[sl], out_hbm.at[c, sl])

    @pl.when(s == NS - 1)
    def _():
        sl = pl.ds((NS - 1) * STRIPE, TAIL)
        pltpu.sync_copy(aggr_s.at[sl], out_hbm.at[c, sl])


@functools.cache
def _get_sc_msg():
  return pl.kernel(
    _sc_msg_body,
    out_type=jax.ShapeDtypeStruct((NC, N, D), jnp.float32),
    mesh=plsc.VectorSubcoreMesh(core_axis_name="c", subcore_axis_name="s"),
    scratch_types=[
        pltpu.VMEM_SHARED((N, D), jnp.float32),
        pltpu.VMEM((EPT,), jnp.int32),
        pltpu.SemaphoreType.DMA((RING,)),
        pltpu.SemaphoreType.DMA((RING,)),
    ] + [pltpu.VMEM((SW, D), jnp.float32) for _ in range(RING)]
      + [pltpu.VMEM((SW,), jnp.int32) for _ in range(RING)]
      + [pltpu.VMEM((SW,), jnp.int32) for _ in range(RING)],
    name="gin_msg_sc",
  )


def _enc_body(xf_ref, a0_ref, a1_ref, bond0_ref, bond1_ref, src_ref,
              dst_ref, e0_ref, e1_ref, e2_ref, h0_ref, combos_ref,
              packed_ref):
    gidx = (src_ref[...] * 8 + e0_ref[...] * 4
            + e1_ref[...] * 2 + e2_ref[...])
    packed_ref[...] = lax.bitwise_or(gidx, lax.shift_left(dst_ref[...], 17))
    xf = xf_ref[...]
    h = jnp.zeros((N, D), jnp.float32)
    for i in range(9):
        cond = xf[:, i:i + 1] > 0.5
        h = h + jnp.where(cond, a1_ref[i], a0_ref[i])
    h0_ref[...] = h
    for l in range(NL):
        rows = []
        for code in range(8):
            t0 = bond1_ref[l, 0] if code & 4 else bond0_ref[l, 0]
            t1 = bond1_ref[l, 1] if code & 2 else bond0_ref[l, 1]
            t2 = bond1_ref[l, 2] if code & 1 else bond0_ref[l, 2]
            rows.append((t0 + t1) + t2)
        combos_ref[l] = jnp.stack(rows)


_enc = pl.pallas_call(
    _enc_body,
    out_shape=(
        jax.ShapeDtypeStruct((N, D), jnp.float32),
        jax.ShapeDtypeStruct((NL, 8, D), jnp.float32),
        jax.ShapeDtypeStruct((E // 128, 128), jnp.int32),
    ),
    name="gin_encoder_tc",
)


def _mlp_body(relu_out, h_ref, a0_ref, a1_ref, eps_ref, w1_ref, b1_ref,
              g1_ref, t1_ref, w2_ref, b2_ref, g2_ref, t2_ref, out_ref):
    z = (1.0 + eps_ref[0, 0]) * h_ref[...] + a0_ref[...] + a1_ref[...]
    u = jnp.dot(z, w1_ref[...], preferred_element_type=jnp.float32)
    u = u + b1_ref[...]
    m = jnp.mean(u, axis=0, keepdims=True)
    u = u - m
    v = jnp.mean(u * u, axis=0, keepdims=True)
    u = u * lax.rsqrt(v + 1e-5) * g1_ref[...] + t1_ref[...]
    u = jnp.maximum(u, 0.0)
    w = jnp.dot(u, w2_ref[...], preferred_element_type=jnp.float32)
    w = w + b2_ref[...]
    m2 = jnp.mean(w, axis=0, keepdims=True)
    w = w - m2
    v2 = jnp.mean(w * w, axis=0, keepdims=True)
    w = w * lax.rsqrt(v2 + 1e-5) * g2_ref[...] + t2_ref[...]
    if relu_out:
        w = jnp.maximum(w, 0.0)
    out_ref[...] = w


_mlp_relu = pl.pallas_call(
    functools.partial(_mlp_body, True),
    out_shape=jax.ShapeDtypeStruct((N, D), jnp.float32),
    name="gin_mlp_relu_tc",
)
_mlp_last = pl.pallas_call(
    functools.partial(_mlp_body, False),
    out_shape=jax.ShapeDtypeStruct((N, D), jnp.float32),
    name="gin_mlp_last_tc",
)


def _expand_body(h_ref, combos_ref, out_ref):
    out_ref[...] = jnp.maximum(
        h_ref[...][:, None, :] + combos_ref[...][None, :, :], 0.0)


_EXB = 1000
_expand = pl.pallas_call(
    _expand_body,
    grid=(N // _EXB,),
    in_specs=[
        pl.BlockSpec((_EXB, D), lambda i: (i, 0)),
        pl.BlockSpec((8, D), lambda i: (0, 0)),
    ],
    out_specs=pl.BlockSpec((_EXB, 8, D), lambda i: (i, 0, 0)),
    out_shape=jax.ShapeDtypeStruct((N, 8, D), jnp.float32),
    name="gin_expand_tc",
)


def kernel(x, edge_index, edge_attr, params):
    xf = jnp.pad(x.astype(jnp.float32), ((0, 0), (0, 16 - x.shape[1])))
    a0 = jnp.pad(jnp.stack([t[0] for t in params["atom_tables"]]),
                 ((0, 16 - len(params["atom_tables"])), (0, 0)))
    a1 = jnp.pad(jnp.stack([t[1] for t in params["atom_tables"]]),
                 ((0, 16 - len(params["atom_tables"])), (0, 0)))
    layers = params["layers"]
    bond0 = jnp.stack([jnp.stack([t[0] for t in p["bond_tables"]])
                       for p in layers])
    bond1 = jnp.stack([jnp.stack([t[1] for t in p["bond_tables"]])
                       for p in layers])

    h0, combos, packed2d = _enc(
        xf, a0, a1, bond0, bond1,
        edge_index[0].reshape(E // 128, 128),
        edge_index[1].reshape(E // 128, 128),
        edge_attr[:, 0].reshape(E // 128, 128),
        edge_attr[:, 1].reshape(E // 128, 128),
        edge_attr[:, 2].reshape(E // 128, 128))
    packed = packed2d.reshape(E)
    zeros = jnp.zeros((STRIPE, D), jnp.float32)

    h = h0
    h2 = _expand(h0, combos[0]).reshape(8 * N, D)
    for l, p in enumerate(layers):
        aggr2 = _get_sc_msg()(h2, packed, zeros)
        args = (h, aggr2[0], aggr2[1], p["eps"].reshape(1, 1),
                p["W1"], p["b1"].reshape(1, -1), p["g1"].reshape(1, -1),
                p["bt1"].reshape(1, -1),
                p["W2"], p["b2"].reshape(1, -1), p["bn_g"].reshape(1, -1),
                p["bn_b"].reshape(1, -1))
        if l < NL - 1:
            h = _mlp_relu(*args)
            h2 = _expand(h, combos[l + 1]).reshape(8 * N, D)
        else:
            h = _mlp_last(*args)
    return h
